# Initial kernel scaffold; baseline (speedup 1.0000x reference)
import jax
import jax.numpy as jnp
from jax.experimental import pallas as pl


def kernel(h, edge_index):
    raise NotImplementedError("write your pallas kernel here")


# speedup vs baseline: 3.4214x; 3.4214x over previous
import functools

import jax
import jax.numpy as jnp
from jax import lax
from jax.experimental import pallas as pl
from jax.experimental.pallas import tpu as pltpu
from jax.experimental.pallas import tpu_sc as plsc

NC = 2
NS = 16
NW = NC * NS
L = 16


def _scores_body(E, D, EPW, CHUNK, NCH,
                 h_hbm, src_hbm, dst_hbm, out_hbm,
                 sidx_v, didx_v, srows_v, drows_v, tmp_v, outc_v,
                 sem_s, sem_d):
    wid = lax.axis_index("s") * NC + lax.axis_index("c")
    base = wid * EPW
    lanes_x16 = lax.iota(jnp.int32, L) * L

    def chunk_body(g, carry):
        off = base + g * CHUNK
        pltpu.sync_copy(src_hbm.at[pl.ds(off, CHUNK)], sidx_v)
        pltpu.sync_copy(dst_hbm.at[pl.ds(off, CHUNK)], didx_v)
        cp_s = pltpu.async_copy(h_hbm.at[sidx_v], srows_v, sem_s)
        cp_d = pltpu.async_copy(h_hbm.at[didx_v], drows_v, sem_d)
        cp_s.wait()
        cp_d.wait()

        def group_body(j, c2):
            jb = j * L
            for jj in range(L):
                e = jb + jj
                a = srows_v[e, pl.ds(0, L)] * drows_v[e, pl.ds(0, L)]
                for cc in range(1, D // L):
                    a = a + (srows_v[e, pl.ds(cc * L, L)]
                             * drows_v[e, pl.ds(cc * L, L)])
                tmp_v[pl.ds(jj * L, L)] = a
            r = plsc.load_gather(tmp_v, [lanes_x16])
            for l in range(1, L):
                r = r + plsc.load_gather(tmp_v, [lanes_x16 + l])
            r = 1.0 / (1.0 + jnp.exp(-r))
            outc_v[pl.ds(jb, L)] = r
            return c2

        lax.fori_loop(0, CHUNK // L, group_body, 0, unroll=False)
        pltpu.sync_copy(outc_v, out_hbm.at[pl.ds(off, CHUNK)])
        return carry

    lax.fori_loop(0, NCH, chunk_body, 0, unroll=False)


def kernel(h, edge_index):
    N, D = h.shape
    E = edge_index.shape[1]
    EPW = E // NW
    CHUNK = 80
    NCH = EPW // CHUNK
    assert EPW * NW == E and NCH * CHUNK == EPW and D % L == 0

    src = edge_index[0]
    dst = edge_index[1]

    mesh = plsc.VectorSubcoreMesh(core_axis_name="c", subcore_axis_name="s",
                                  num_cores=NC, num_subcores=NS)
    body = functools.partial(_scores_body, E, D, EPW, CHUNK, NCH)
    f = pl.kernel(
        body,
        out_type=jax.ShapeDtypeStruct((E,), jnp.float32),
        mesh=mesh,
        compiler_params=pltpu.CompilerParams(needs_layout_passes=False),
        scratch_types=[
            pltpu.VMEM((CHUNK,), jnp.int32),
            pltpu.VMEM((CHUNK,), jnp.int32),
            pltpu.VMEM((CHUNK, D), jnp.float32),
            pltpu.VMEM((CHUNK, D), jnp.float32),
            pltpu.VMEM((L * L,), jnp.float32),
            pltpu.VMEM((CHUNK,), jnp.float32),
            pltpu.SemaphoreType.DMA,
            pltpu.SemaphoreType.DMA,
        ],
    )
    return f(h, src, dst)


# speedup vs baseline: 7.7300x; 2.2593x over previous
import functools

import jax
import jax.numpy as jnp
from jax import lax
from jax.experimental import pallas as pl
from jax.experimental.pallas import tpu as pltpu
from jax.experimental.pallas import tpu_sc as plsc

NC = 2
NS = 16
NW = NC * NS
L = 16


def _scores_body(E, D, EPW, CHUNK, NCH,
                 h_hbm, src_hbm, dst_hbm, out_hbm,
                 sidx_v, didx_v, srows0, drows0, srows1, drows1,
                 tmp_v, outall_v,
                 sem_s0, sem_d0, sem_s1, sem_d1):
    wid = lax.axis_index("s") * NC + lax.axis_index("c")
    base = wid * EPW
    lanes_x16 = lax.iota(jnp.int32, L) * L
    bufs = ((srows0, drows0, sem_s0, sem_d0),
            (srows1, drows1, sem_s1, sem_d1))

    def issue(g, srows, drows, sem_s, sem_d):
        pltpu.async_copy(h_hbm.at[sidx_v.at[pl.ds(g * CHUNK, CHUNK)]],
                         srows, sem_s)
        pltpu.async_copy(h_hbm.at[didx_v.at[pl.ds(g * CHUNK, CHUNK)]],
                         drows, sem_d)

    def wait(g, srows, drows, sem_s, sem_d):
        pltpu.make_async_copy(h_hbm.at[sidx_v.at[pl.ds(g * CHUNK, CHUNK)]],
                              srows, sem_s).wait()
        pltpu.make_async_copy(h_hbm.at[didx_v.at[pl.ds(g * CHUNK, CHUNK)]],
                              drows, sem_d).wait()

    def compute(g, srows, drows):
        def group_body(j, c2):
            jb = j * L
            for jj in range(L):
                e = jb + jj
                a = srows[e, pl.ds(0, L)] * drows[e, pl.ds(0, L)]
                for cc in range(1, D // L):
                    a = a + (srows[e, pl.ds(cc * L, L)]
                             * drows[e, pl.ds(cc * L, L)])
                tmp_v[pl.ds(jj * L, L)] = a
            r = plsc.load_gather(tmp_v, [lanes_x16])
            for l in range(1, L):
                r = r + plsc.load_gather(tmp_v, [lanes_x16 + l])
            r = 1.0 / (1.0 + jnp.exp(-r))
            outall_v[pl.ds(g * CHUNK + jb, L)] = r
            return c2

        lax.fori_loop(0, CHUNK // L, group_body, 0, unroll=False)

    pltpu.sync_copy(src_hbm.at[pl.ds(base, EPW)], sidx_v)
    pltpu.sync_copy(dst_hbm.at[pl.ds(base, EPW)], didx_v)

    issue(0, *bufs[0])

    def pair_body(g2, carry):
        g = g2 * 2
        for b in range(2):
            gg = g + b
            wait(gg, *bufs[b])
            issue(gg + 1, *bufs[1 - b])
            compute(gg, bufs[b][0], bufs[b][1])
        return carry

    lax.fori_loop(0, (NCH - 1) // 2, pair_body, 0, unroll=False)
    last = NCH - 1
    wait(last, *bufs[last % 2])
    compute(last, bufs[last % 2][0], bufs[last % 2][1])

    pltpu.sync_copy(outall_v, out_hbm.at[pl.ds(base, EPW)])


def kernel(h, edge_index):
    N, D = h.shape
    E = edge_index.shape[1]
    EPW = E // NW
    CHUNK = 80
    NCH = EPW // CHUNK
    assert EPW * NW == E and NCH * CHUNK == EPW and D % L == 0
    assert NCH % 2 == 1

    src = edge_index[0]
    dst = edge_index[1]

    mesh = plsc.VectorSubcoreMesh(core_axis_name="c", subcore_axis_name="s",
                                  num_cores=NC, num_subcores=NS)
    body = functools.partial(_scores_body, E, D, EPW, CHUNK, NCH)
    f = pl.kernel(
        body,
        out_type=jax.ShapeDtypeStruct((E,), jnp.float32),
        mesh=mesh,
        compiler_params=pltpu.CompilerParams(needs_layout_passes=False),
        scratch_types=[
            pltpu.VMEM((EPW,), jnp.int32),
            pltpu.VMEM((EPW,), jnp.int32),
            pltpu.VMEM((CHUNK, D), jnp.float32),
            pltpu.VMEM((CHUNK, D), jnp.float32),
            pltpu.VMEM((CHUNK, D), jnp.float32),
            pltpu.VMEM((CHUNK, D), jnp.float32),
            pltpu.VMEM((L * L,), jnp.float32),
            pltpu.VMEM((EPW,), jnp.float32),
            pltpu.SemaphoreType.DMA,
            pltpu.SemaphoreType.DMA,
            pltpu.SemaphoreType.DMA,
            pltpu.SemaphoreType.DMA,
        ],
    )
    return f(h, src, dst)
